# baseline (speedup 1.0000x reference)
import functools

import jax
import jax.numpy as jnp
from jax.experimental import pallas as pl
from jax.experimental.pallas import tpu as pltpu

_BV = 2048


def _proj_block(x_ref, w_ref, b_ref, o_ref):
    acc = jax.lax.dot_general(
        x_ref[...],
        w_ref[...],
        dimension_numbers=(((1,), (1,)), ((), ())),
        preferred_element_type=jnp.float32,
    )
    o_ref[...] = acc + b_ref[...]


@functools.partial(jax.jit, static_argnames=())
def _logits(inputs, W, b):
    batch, nhid = inputs.shape
    ntokens = W.shape[0]
    b2 = b.reshape(1, ntokens)
    grid = (pl.cdiv(ntokens, _BV),)
    return pl.pallas_call(
        _proj_block,
        grid=grid,
        in_specs=[
            pl.BlockSpec((batch, nhid), lambda i: (0, 0)),
            pl.BlockSpec((_BV, nhid), lambda i: (i, 0)),
            pl.BlockSpec((1, _BV), lambda i: (0, i)),
        ],
        out_specs=pl.BlockSpec((batch, _BV), lambda i: (0, i)),
        out_shape=jax.ShapeDtypeStruct((batch, ntokens), jnp.float32),
        compiler_params=pltpu.CompilerParams(
            dimension_semantics=("parallel",),
        ),
    )(inputs, W, b2)


def kernel(inputs, labels, W, b):
    return (_logits(inputs, W, b), labels)


# speedup vs baseline: 1.0044x; 1.0044x over previous
import functools

import jax
import jax.numpy as jnp
from jax.experimental import pallas as pl
from jax.experimental.pallas import tpu as pltpu

_BV = 4096


def _proj_block(x_ref, w_ref, b_ref, o_ref):
    acc = jax.lax.dot_general(
        x_ref[...],
        w_ref[...],
        dimension_numbers=(((1,), (1,)), ((), ())),
        preferred_element_type=jnp.float32,
    )
    o_ref[...] = acc + b_ref[...]


@functools.partial(jax.jit, static_argnames=())
def _logits(inputs, W, b):
    batch, nhid = inputs.shape
    ntokens = W.shape[0]
    b2 = b.reshape(1, ntokens)
    grid = (pl.cdiv(ntokens, _BV),)
    return pl.pallas_call(
        _proj_block,
        grid=grid,
        in_specs=[
            pl.BlockSpec((batch, nhid), lambda i: (0, 0)),
            pl.BlockSpec((_BV, nhid), lambda i: (i, 0)),
            pl.BlockSpec((1, _BV), lambda i: (0, i)),
        ],
        out_specs=pl.BlockSpec((batch, _BV), lambda i: (0, i)),
        out_shape=jax.ShapeDtypeStruct((batch, ntokens), jnp.float32),
        compiler_params=pltpu.CompilerParams(
            dimension_semantics=("parallel",),
        ),
    )(inputs, W, b2)


def kernel(inputs, labels, W, b):
    return (_logits(inputs, W, b), labels)
